# baseline (speedup 1.0000x reference)
import functools

import jax
import jax.numpy as jnp
from jax import lax
from jax.experimental import pallas as pl
from jax.experimental.pallas import tpu as pltpu
from jax.experimental.pallas import tpu_sc as plsc

ALPHA = 0.25
BETA = 0.25

RB = 512
CB = 2048


def _k1_body(x_ref, cb_ref, zn_ref, cn_ref, idx_ref, cnt_ref, *, num_codes: int):
    i = pl.program_id(0)
    x = x_ref[...]
    zn = zn_ref[...][:, 0]
    nchunks = num_codes // CB

    def step(j, carry):
        bd, bi = carry
        c = cb_ref[pl.ds(j * CB, CB), :]
        zc = lax.dot_general(
            x, c, (((1,), (1,)), ((), ())),
            preferred_element_type=jnp.float32,
            precision=lax.Precision.DEFAULT)

        cn = cn_ref[0, pl.ds(j * CB, CB)]
        s = (zn[:, None] - 2.0 * zc) + cn[None, :]
        m = jnp.min(s, axis=1)
        ii = lax.broadcasted_iota(jnp.int32, (RB, CB), 1)
        cand = jnp.where(s == m[:, None], ii, num_codes)
        a = jnp.min(cand, axis=1) + j * CB
        take = (m < bd) | ((m == bd) & (a < bi))
        nbd = jnp.where(take, m, bd)
        nbd = nbd.astype(jnp.bfloat16).astype(jnp.float32)
        return nbd, jnp.where(take, a, bi)

    init = (jnp.full((RB,), jnp.inf, jnp.float32),
            jnp.zeros((RB,), jnp.int32))
    _, bi = lax.fori_loop(0, nchunks, step, init)
    idx_ref[...] = bi

    for j2 in range(nchunks):
        ii = lax.broadcasted_iota(jnp.int32, (RB, CB), 1) + j2 * CB
        ohs = jnp.sum((bi[:, None] == ii).astype(jnp.float32), axis=0)
        prev = jnp.where(i == 0, 0.0, cnt_ref[pl.ds(j2 * CB, CB)])
        cnt_ref[pl.ds(j2 * CB, CB)] = prev + ohs


def _k1(flat, codebook, zn, cn):
    n, d = flat.shape
    num_codes = codebook.shape[0]
    grid = (n // RB,)
    return pl.pallas_call(
        functools.partial(_k1_body, num_codes=num_codes),
        grid=grid,
        in_specs=[
            pl.BlockSpec((RB, d), lambda i: (i, 0)),
            pl.BlockSpec((num_codes, d), lambda i: (0, 0)),
            pl.BlockSpec((RB, 1), lambda i: (i, 0)),
            pl.BlockSpec((1, num_codes), lambda i: (0, 0)),
        ],
        out_specs=[
            pl.BlockSpec((RB,), lambda i: (i,)),
            pl.BlockSpec((num_codes,), lambda i: (0,)),
        ],
        out_shape=[
            jax.ShapeDtypeStruct((n,), jnp.int32),
            jax.ShapeDtypeStruct((num_codes,), jnp.float32),
        ],
    )(flat, codebook, zn, cn)


_NC, _NS, _L = 2, 16, 16
_NW = _NC * _NS
_ROWS_PER_TXN = 128


def _make_k2(n, d, num_codes):
    bpw = n // _NW
    ntxn = bpw // _ROWS_PER_TXN
    mesh = plsc.VectorSubcoreMesh(core_axis_name="c", subcore_axis_name="s")

    @functools.partial(
        pl.kernel,
        out_type=jax.ShapeDtypeStruct((n, d), jnp.float32),
        mesh=mesh,
        scratch_types=[
            [pltpu.VMEM((_ROWS_PER_TXN,), jnp.int32) for _ in range(ntxn)],
            [pltpu.VMEM((_ROWS_PER_TXN, d), jnp.float32) for _ in range(ntxn)],
            pltpu.SemaphoreType.DMA,
        ],
    )
    def k2(idx_hbm, cb_hbm, zq_hbm, idx_vs, rows_vs, sem):
        c = lax.axis_index("c")
        s = lax.axis_index("s")
        wid = s * _NC + c
        base = wid * bpw

        for j in range(ntxn):
            pltpu.sync_copy(idx_hbm.at[wid * ntxn + j], idx_vs[j])
        for j in range(ntxn):
            pltpu.async_copy(cb_hbm.at[idx_vs[j]], rows_vs[j], sem).wait()
            pltpu.sync_copy(rows_vs[j],
                            zq_hbm.at[pl.ds(base + j * _ROWS_PER_TXN,
                                            _ROWS_PER_TXN)])

    return k2


K3_RB = 1024


def _k3_body(ze_ref, zq_ref, cnt_ref, zqst_ref, loss_ref, ppl_ref, acc_ref,
             *, nsteps: int, n: int, d: int):
    i = pl.program_id(0)
    ze = ze_ref[...]
    zq = zq_ref[...]
    zqst_ref[...] = ze + (zq - ze)
    blk = jnp.sum((ze - zq) ** 2)

    @pl.when(i == 0)
    def _():
        acc_ref[0, 0] = blk

    @pl.when(i > 0)
    def _():
        acc_ref[0, 0] = acc_ref[0, 0] + blk

    @pl.when(i == nsteps - 1)
    def _():
        m = acc_ref[0, 0] / jnp.float32(n * d)
        loss_ref[...] = jnp.reshape(ALPHA * (m + BETA * m), (1, 1))
        p = cnt_ref[...] / jnp.float32(n)
        ent = jnp.sum(p * jnp.log(p + 1e-10))
        ppl_ref[...] = jnp.reshape(jnp.exp(-ent), (1, 1))


def _k3(flat, zq, counts):
    n, d = flat.shape
    num_codes = counts.shape[0]
    nsteps = n // K3_RB
    return pl.pallas_call(
        functools.partial(_k3_body, nsteps=nsteps, n=n, d=d),
        grid=(nsteps,),
        in_specs=[
            pl.BlockSpec((K3_RB, d), lambda i: (i, 0)),
            pl.BlockSpec((K3_RB, d), lambda i: (i, 0)),
            pl.BlockSpec((num_codes,), lambda i: (0,)),
        ],
        out_specs=[
            pl.BlockSpec((K3_RB, d), lambda i: (i, 0)),
            pl.BlockSpec((1, 1), lambda i: (0, 0)),
            pl.BlockSpec((1, 1), lambda i: (0, 0)),
        ],
        out_shape=[
            jax.ShapeDtypeStruct((n, d), jnp.float32),
            jax.ShapeDtypeStruct((1, 1), jnp.float32),
            jax.ShapeDtypeStruct((1, 1), jnp.float32),
        ],
        scratch_shapes=[pltpu.SMEM((1, 1), jnp.float32)],
    )(flat, zq, counts)


def kernel(z_e, codebook):
    b, nseq, d = z_e.shape
    n = b * nseq
    num_codes = codebook.shape[0]
    flat = z_e.reshape(n, d)
    zn = jnp.sum(flat * flat, axis=1, keepdims=True)
    cn = jnp.sum(codebook * codebook, axis=1)[None, :]

    idx, counts = _k1(flat, codebook, zn, cn)
    zq = _make_k2(n, d, num_codes)(idx.reshape(n // 128, 128), codebook)
    zqst, loss, ppl = _k3(flat, zq, counts)
    return (zqst.reshape(b, nseq, d),
            loss.reshape(()),
            ppl.reshape(()))


# speedup vs baseline: 1.0165x; 1.0165x over previous
import functools

import jax
import jax.numpy as jnp
from jax import lax
from jax.experimental import pallas as pl
from jax.experimental.pallas import tpu as pltpu
from jax.experimental.pallas import tpu_sc as plsc

ALPHA = 0.25
BETA = 0.25

RB = 512
CB = 2048


def _k1_body(x_ref, cb_ref, zn_ref, cn_ref, idx_ref, *, num_codes: int):
    xm2 = x_ref[...] * -2.0
    zn = zn_ref[...][:, 0]
    nchunks = num_codes // CB

    def step(j, carry):
        bd, bi = carry
        c = cb_ref[pl.ds(j * CB, CB), :]
        zc = lax.dot_general(
            xm2, c, (((1,), (1,)), ((), ())),
            preferred_element_type=jnp.float32,
            precision=lax.Precision.DEFAULT)

        cn = cn_ref[0, pl.ds(j * CB, CB)]
        s = (zn[:, None] + zc) + cn[None, :]
        m = jnp.min(s, axis=1)
        ii = lax.broadcasted_iota(jnp.int32, (RB, CB), 1)
        cand = jnp.where(s == m[:, None], ii, num_codes)
        a = jnp.min(cand, axis=1) + j * CB
        take = (m < bd) | ((m == bd) & (a < bi))
        nbd = jnp.where(take, m, bd)
        nbd = nbd.astype(jnp.bfloat16).astype(jnp.float32)
        return nbd, jnp.where(take, a, bi)

    init = (jnp.full((RB,), jnp.inf, jnp.float32),
            jnp.zeros((RB,), jnp.int32))
    _, bi = lax.fori_loop(0, nchunks, step, init)
    idx_ref[...] = bi


def _k1(flat, codebook, zn, cn):
    n, d = flat.shape
    num_codes = codebook.shape[0]
    grid = (n // RB,)
    return pl.pallas_call(
        functools.partial(_k1_body, num_codes=num_codes),
        grid=grid,
        in_specs=[
            pl.BlockSpec((RB, d), lambda i: (i, 0)),
            pl.BlockSpec((num_codes, d), lambda i: (0, 0)),
            pl.BlockSpec((RB, 1), lambda i: (i, 0)),
            pl.BlockSpec((1, num_codes), lambda i: (0, 0)),
        ],
        out_specs=pl.BlockSpec((RB,), lambda i: (i,)),
        out_shape=jax.ShapeDtypeStruct((n,), jnp.int32),
    )(flat, codebook, zn, cn)


K4_CB = 2048


K4_RB = 512


def _k4_body(idx_ref, ppl_ref, *, n: int, num_codes: int):
    ent = jnp.float32(0.0)
    inv_n = 1.0 / jnp.float32(n)
    for j in range(num_codes // K4_CB):
        codes = lax.broadcasted_iota(jnp.int32, (K4_RB, K4_CB), 1) + j * K4_CB
        cnt = jnp.zeros((K4_CB,), jnp.float32)
        for k in range(n // K4_RB):
            bi = idx_ref[pl.ds(k * K4_RB, K4_RB)]
            cnt = cnt + jnp.sum((bi[:, None] == codes).astype(jnp.float32),
                                axis=0)
        p = cnt * inv_n
        ent = ent + jnp.sum(p * jnp.log(p + 1e-10))
    ppl_ref[...] = jnp.reshape(jnp.exp(-ent), (1, 1))


def _k4(idx, n, num_codes):
    return pl.pallas_call(
        functools.partial(_k4_body, n=n, num_codes=num_codes),
        grid=(1,),
        in_specs=[pl.BlockSpec((n,), lambda i: (0,))],
        out_specs=pl.BlockSpec((1, 1), lambda i: (0, 0)),
        out_shape=jax.ShapeDtypeStruct((1, 1), jnp.float32),
    )(idx)


_NC, _NS, _L = 2, 16, 16
_NW = _NC * _NS
_ROWS_PER_TXN = 128


def _make_k2(n, d, num_codes):
    bpw = n // _NW
    ntxn = bpw // _ROWS_PER_TXN
    mesh = plsc.VectorSubcoreMesh(core_axis_name="c", subcore_axis_name="s")

    @functools.partial(
        pl.kernel,
        out_type=jax.ShapeDtypeStruct((n, d), jnp.float32),
        mesh=mesh,
        scratch_types=[
            [pltpu.VMEM((_ROWS_PER_TXN,), jnp.int32) for _ in range(ntxn)],
            [pltpu.VMEM((_ROWS_PER_TXN, d), jnp.float32) for _ in range(ntxn)],
            pltpu.SemaphoreType.DMA,
        ],
    )
    def k2(idx_hbm, cb_hbm, zq_hbm, idx_vs, rows_vs, sem):
        c = lax.axis_index("c")
        s = lax.axis_index("s")
        wid = s * _NC + c
        base = wid * bpw

        for j in range(ntxn):
            pltpu.sync_copy(idx_hbm.at[wid * ntxn + j], idx_vs[j])
        for j in range(ntxn):
            pltpu.async_copy(cb_hbm.at[idx_vs[j]], rows_vs[j], sem).wait()
            pltpu.sync_copy(rows_vs[j],
                            zq_hbm.at[pl.ds(base + j * _ROWS_PER_TXN,
                                            _ROWS_PER_TXN)])

    return k2


K3_RB = 1024


def _k3_body(ze_ref, zq_ref, zqst_ref, loss_ref, acc_ref,
             *, nsteps: int, n: int, d: int):
    i = pl.program_id(0)
    ze = ze_ref[...]
    zq = zq_ref[...]
    zqst_ref[...] = ze + (zq - ze)
    blk = jnp.sum((ze - zq) ** 2)

    @pl.when(i == 0)
    def _():
        acc_ref[0, 0] = blk

    @pl.when(i > 0)
    def _():
        acc_ref[0, 0] = acc_ref[0, 0] + blk

    @pl.when(i == nsteps - 1)
    def _():
        m = acc_ref[0, 0] / jnp.float32(n * d)
        loss_ref[...] = jnp.reshape(ALPHA * (m + BETA * m), (1, 1))


def _k3(flat, zq):
    n, d = flat.shape
    nsteps = n // K3_RB
    return pl.pallas_call(
        functools.partial(_k3_body, nsteps=nsteps, n=n, d=d),
        grid=(nsteps,),
        in_specs=[
            pl.BlockSpec((K3_RB, d), lambda i: (i, 0)),
            pl.BlockSpec((K3_RB, d), lambda i: (i, 0)),
        ],
        out_specs=[
            pl.BlockSpec((K3_RB, d), lambda i: (i, 0)),
            pl.BlockSpec((1, 1), lambda i: (0, 0)),
        ],
        out_shape=[
            jax.ShapeDtypeStruct((n, d), jnp.float32),
            jax.ShapeDtypeStruct((1, 1), jnp.float32),
        ],
        scratch_shapes=[pltpu.SMEM((1, 1), jnp.float32)],
    )(flat, zq)


def kernel(z_e, codebook):
    b, nseq, d = z_e.shape
    n = b * nseq
    num_codes = codebook.shape[0]
    flat = z_e.reshape(n, d)
    zn = jnp.sum(flat * flat, axis=1, keepdims=True)
    cn = jnp.sum(codebook * codebook, axis=1)[None, :]

    idx = _k1(flat, codebook, zn, cn)
    idx2d = idx.reshape(n // 128, 128)
    zq = _make_k2(n, d, num_codes)(idx2d, codebook)
    ppl = _k4(idx, n, num_codes)
    zqst, loss = _k3(flat, zq)
    return (zqst.reshape(b, nseq, d),
            loss.reshape(()),
            ppl.reshape(()))


# speedup vs baseline: 1.0167x; 1.0002x over previous
import functools

import jax
import jax.numpy as jnp
from jax import lax
from jax.experimental import pallas as pl
from jax.experimental.pallas import tpu as pltpu
from jax.experimental.pallas import tpu_sc as plsc

ALPHA = 0.25
BETA = 0.25

RB = 512
CB = 2048


def _k1_body(x_ref, cb_ref, zn_ref, cn_ref, idx_ref, *, num_codes: int):
    xm2 = x_ref[...] * -2.0
    zn = zn_ref[...][:, 0]
    nchunks = num_codes // CB

    def step(j, carry):
        bd, bi = carry
        c = cb_ref[pl.ds(j * CB, CB), :]
        zc = lax.dot_general(
            xm2, c, (((1,), (1,)), ((), ())),
            preferred_element_type=jnp.float32,
            precision=lax.Precision.DEFAULT)

        cn = cn_ref[0, pl.ds(j * CB, CB)]
        s = (zn[:, None] + zc) + cn[None, :]
        m = jnp.min(s, axis=1)
        ii = lax.broadcasted_iota(jnp.int32, (RB, CB), 1)
        cand = jnp.where(s == m[:, None], ii, num_codes)
        a = jnp.min(cand, axis=1) + j * CB
        take = (m < bd) | ((m == bd) & (a < bi))
        nbd = jnp.where(take, m, bd)
        nbd = nbd.astype(jnp.bfloat16).astype(jnp.float32)
        return nbd, jnp.where(take, a, bi)

    init = (jnp.full((RB,), jnp.inf, jnp.float32),
            jnp.zeros((RB,), jnp.int32))
    _, bi = lax.fori_loop(0, nchunks, step, init)
    idx_ref[...] = bi


def _k1(flat, codebook, zn, cn):
    n, d = flat.shape
    num_codes = codebook.shape[0]
    grid = (n // RB,)
    return pl.pallas_call(
        functools.partial(_k1_body, num_codes=num_codes),
        grid=grid,
        in_specs=[
            pl.BlockSpec((RB, d), lambda i: (i, 0)),
            pl.BlockSpec((num_codes, d), lambda i: (0, 0)),
            pl.BlockSpec((RB, 1), lambda i: (i, 0)),
            pl.BlockSpec((1, num_codes), lambda i: (0, 0)),
        ],
        out_specs=pl.BlockSpec((RB,), lambda i: (i,)),
        out_shape=jax.ShapeDtypeStruct((n,), jnp.int32),
    )(flat, codebook, zn, cn)


K4_CB = 2048


K4_RB = 512


_NC, _NS, _L = 2, 16, 16
_NW = _NC * _NS
_ROWS_PER_TXN = 128


def _make_k2(n, d, num_codes):
    bpw = n // _NW
    ntxn = bpw // _ROWS_PER_TXN
    mesh = plsc.VectorSubcoreMesh(core_axis_name="c", subcore_axis_name="s")

    @functools.partial(
        pl.kernel,
        out_type=jax.ShapeDtypeStruct((n, d), jnp.float32),
        mesh=mesh,
        scratch_types=[
            [pltpu.VMEM((_ROWS_PER_TXN,), jnp.int32) for _ in range(ntxn)],
            [pltpu.VMEM((_ROWS_PER_TXN, d), jnp.float32) for _ in range(ntxn)],
            pltpu.SemaphoreType.DMA,
        ],
    )
    def k2(idx_hbm, cb_hbm, zq_hbm, idx_vs, rows_vs, sem):
        c = lax.axis_index("c")
        s = lax.axis_index("s")
        wid = s * _NC + c
        base = wid * bpw

        for j in range(ntxn):
            pltpu.sync_copy(idx_hbm.at[wid * ntxn + j], idx_vs[j])
        for j in range(ntxn):
            pltpu.async_copy(cb_hbm.at[idx_vs[j]], rows_vs[j], sem).wait()
            pltpu.sync_copy(rows_vs[j],
                            zq_hbm.at[pl.ds(base + j * _ROWS_PER_TXN,
                                            _ROWS_PER_TXN)])

    return k2


K3_RB = 1024


def _k3_body(ze_ref, zq_ref, idx_ref, zqst_ref, loss_ref, ppl_ref,
             acc_ref, ent_ref, *, nsteps: int, n: int, d: int,
             num_codes: int):
    i = pl.program_id(0)
    ze = ze_ref[...]
    zq = zq_ref[...]
    zqst_ref[...] = ze + (zq - ze)
    blk = jnp.sum((ze - zq) ** 2)

    ccs = num_codes // nsteps
    codes = lax.broadcasted_iota(jnp.int32, (512, ccs), 1) + i * ccs
    cnt = jnp.zeros((ccs,), jnp.float32)
    for k in range(n // 512):
        bi = idx_ref[pl.ds(k * 512, 512)]
        cnt = cnt + jnp.sum((bi[:, None] == codes).astype(jnp.float32), axis=0)
    p = cnt * (1.0 / jnp.float32(n))
    ent_blk = jnp.sum(p * jnp.log(p + 1e-10))

    @pl.when(i == 0)
    def _():
        acc_ref[0, 0] = blk
        ent_ref[0, 0] = ent_blk

    @pl.when(i > 0)
    def _():
        acc_ref[0, 0] = acc_ref[0, 0] + blk
        ent_ref[0, 0] = ent_ref[0, 0] + ent_blk

    @pl.when(i == nsteps - 1)
    def _():
        m = acc_ref[0, 0] / jnp.float32(n * d)
        loss_ref[...] = jnp.reshape(ALPHA * (m + BETA * m), (1, 1))
        ppl_ref[...] = jnp.reshape(jnp.exp(-(ent_ref[0, 0])), (1, 1))


def _k3(flat, zq, idx):
    n, d = flat.shape
    num_codes = 8192
    nsteps = n // K3_RB
    return pl.pallas_call(
        functools.partial(_k3_body, nsteps=nsteps, n=n, d=d,
                          num_codes=num_codes),
        grid=(nsteps,),
        in_specs=[
            pl.BlockSpec((K3_RB, d), lambda i: (i, 0)),
            pl.BlockSpec((K3_RB, d), lambda i: (i, 0)),
            pl.BlockSpec((n,), lambda i: (0,)),
        ],
        out_specs=[
            pl.BlockSpec((K3_RB, d), lambda i: (i, 0)),
            pl.BlockSpec((1, 1), lambda i: (0, 0)),
            pl.BlockSpec((1, 1), lambda i: (0, 0)),
        ],
        out_shape=[
            jax.ShapeDtypeStruct((n, d), jnp.float32),
            jax.ShapeDtypeStruct((1, 1), jnp.float32),
            jax.ShapeDtypeStruct((1, 1), jnp.float32),
        ],
        scratch_shapes=[pltpu.SMEM((1, 1), jnp.float32),
                        pltpu.SMEM((1, 1), jnp.float32)],
    )(flat, zq, idx)


def kernel(z_e, codebook):
    b, nseq, d = z_e.shape
    n = b * nseq
    num_codes = codebook.shape[0]
    flat = z_e.reshape(n, d)
    zn = jnp.sum(flat * flat, axis=1, keepdims=True)
    cn = jnp.sum(codebook * codebook, axis=1)[None, :]

    idx = _k1(flat, codebook, zn, cn)
    idx2d = idx.reshape(n // 128, 128)
    zq = _make_k2(n, d, num_codes)(idx2d, codebook)
    zqst, loss, ppl = _k3(flat, zq, idx)
    return (zqst.reshape(b, nseq, d),
            loss.reshape(()),
            ppl.reshape(()))


# speedup vs baseline: 1.0804x; 1.0627x over previous
import functools

import jax
import jax.numpy as jnp
from jax import lax
from jax.experimental import pallas as pl
from jax.experimental.pallas import tpu as pltpu
from jax.experimental.pallas import tpu_sc as plsc

ALPHA = 0.25
BETA = 0.25

RB = 1024
CB = 2048


def _k1_body(x_ref, cb_ref, zn_ref, cn_ref, idx_ref, *, num_codes: int):
    xm2 = x_ref[...] * -2.0
    zn = zn_ref[...][:, 0]
    nchunks = num_codes // CB

    def step(j, carry):
        bd, bi = carry
        c = cb_ref[pl.ds(j * CB, CB), :]
        zc = lax.dot_general(
            xm2, c, (((1,), (1,)), ((), ())),
            preferred_element_type=jnp.float32,
            precision=lax.Precision.DEFAULT)

        cn = cn_ref[0, pl.ds(j * CB, CB)]
        s = (zn[:, None] + zc) + cn[None, :]
        m = jnp.min(s, axis=1)
        ii = lax.broadcasted_iota(jnp.int32, (RB, CB), 1)
        cand = jnp.where(s == m[:, None], ii, num_codes)
        a = jnp.min(cand, axis=1) + j * CB
        take = (m < bd) | ((m == bd) & (a < bi))
        nbd = jnp.where(take, m, bd)
        nbd = nbd.astype(jnp.bfloat16).astype(jnp.float32)
        return nbd, jnp.where(take, a, bi)

    init = (jnp.full((RB,), jnp.inf, jnp.float32),
            jnp.zeros((RB,), jnp.int32))
    _, bi = lax.fori_loop(0, nchunks, step, init)
    idx_ref[...] = bi


def _k1(flat, codebook, zn, cn):
    n, d = flat.shape
    num_codes = codebook.shape[0]
    grid = (n // RB,)
    return pl.pallas_call(
        functools.partial(_k1_body, num_codes=num_codes),
        grid=grid,
        in_specs=[
            pl.BlockSpec((RB, d), lambda i: (i, 0)),
            pl.BlockSpec((num_codes, d), lambda i: (0, 0)),
            pl.BlockSpec((RB, 1), lambda i: (i, 0)),
            pl.BlockSpec((1, num_codes), lambda i: (0, 0)),
        ],
        out_specs=pl.BlockSpec((RB,), lambda i: (i,)),
        out_shape=jax.ShapeDtypeStruct((n,), jnp.int32),
    )(flat, codebook, zn, cn)


K4_CB = 2048


K4_RB = 512


_NC, _NS, _L = 2, 16, 16
_NW = _NC * _NS
_ROWS_PER_TXN = 128


def _make_k2(n, d, num_codes):
    bpw = n // _NW
    ntxn = bpw // _ROWS_PER_TXN
    mesh = plsc.VectorSubcoreMesh(core_axis_name="c", subcore_axis_name="s")

    @functools.partial(
        pl.kernel,
        out_type=jax.ShapeDtypeStruct((n, d), jnp.float32),
        mesh=mesh,
        scratch_types=[
            [pltpu.VMEM((_ROWS_PER_TXN,), jnp.int32) for _ in range(ntxn)],
            [pltpu.VMEM((_ROWS_PER_TXN, d), jnp.float32) for _ in range(ntxn)],
            pltpu.SemaphoreType.DMA,
        ],
    )
    def k2(idx_hbm, cb_hbm, zq_hbm, idx_vs, rows_vs, sem):
        c = lax.axis_index("c")
        s = lax.axis_index("s")
        wid = s * _NC + c
        base = wid * bpw

        for j in range(ntxn):
            pltpu.sync_copy(idx_hbm.at[wid * ntxn + j], idx_vs[j])
        copies = [pltpu.async_copy(cb_hbm.at[idx_vs[j]], rows_vs[j], sem)
                  for j in range(ntxn)]
        for j in range(ntxn):
            copies[j].wait()
            pltpu.sync_copy(rows_vs[j],
                            zq_hbm.at[pl.ds(base + j * _ROWS_PER_TXN,
                                            _ROWS_PER_TXN)])

    return k2


K3_RB = 1024


def _k3_body(ze_ref, zq_ref, idx_ref, zqst_ref, loss_ref, ppl_ref,
             acc_ref, ent_ref, *, nsteps: int, n: int, d: int,
             num_codes: int):
    i = pl.program_id(0)
    ze = ze_ref[...]
    zq = zq_ref[...]
    zqst_ref[...] = ze + (zq - ze)
    blk = jnp.sum((ze - zq) ** 2)

    ccs = num_codes // nsteps
    codes = lax.broadcasted_iota(jnp.int32, (512, ccs), 1) + i * ccs
    cnt = jnp.zeros((ccs,), jnp.float32)
    for k in range(n // 512):
        bi = idx_ref[pl.ds(k * 512, 512)]
        cnt = cnt + jnp.sum((bi[:, None] == codes).astype(jnp.float32), axis=0)
    p = cnt * (1.0 / jnp.float32(n))
    ent_blk = jnp.sum(p * jnp.log(p + 1e-10))

    @pl.when(i == 0)
    def _():
        acc_ref[0, 0] = blk
        ent_ref[0, 0] = ent_blk

    @pl.when(i > 0)
    def _():
        acc_ref[0, 0] = acc_ref[0, 0] + blk
        ent_ref[0, 0] = ent_ref[0, 0] + ent_blk

    @pl.when(i == nsteps - 1)
    def _():
        m = acc_ref[0, 0] / jnp.float32(n * d)
        loss_ref[...] = jnp.reshape(ALPHA * (m + BETA * m), (1, 1))
        ppl_ref[...] = jnp.reshape(jnp.exp(-(ent_ref[0, 0])), (1, 1))


def _k3(flat, zq, idx):
    n, d = flat.shape
    num_codes = 8192
    nsteps = n // K3_RB
    return pl.pallas_call(
        functools.partial(_k3_body, nsteps=nsteps, n=n, d=d,
                          num_codes=num_codes),
        grid=(nsteps,),
        in_specs=[
            pl.BlockSpec((K3_RB, d), lambda i: (i, 0)),
            pl.BlockSpec((K3_RB, d), lambda i: (i, 0)),
            pl.BlockSpec((n,), lambda i: (0,)),
        ],
        out_specs=[
            pl.BlockSpec((K3_RB, d), lambda i: (i, 0)),
            pl.BlockSpec((1, 1), lambda i: (0, 0)),
            pl.BlockSpec((1, 1), lambda i: (0, 0)),
        ],
        out_shape=[
            jax.ShapeDtypeStruct((n, d), jnp.float32),
            jax.ShapeDtypeStruct((1, 1), jnp.float32),
            jax.ShapeDtypeStruct((1, 1), jnp.float32),
        ],
        scratch_shapes=[pltpu.SMEM((1, 1), jnp.float32),
                        pltpu.SMEM((1, 1), jnp.float32)],
    )(flat, zq, idx)


def kernel(z_e, codebook):
    b, nseq, d = z_e.shape
    n = b * nseq
    num_codes = codebook.shape[0]
    flat = z_e.reshape(n, d)
    zn = jnp.sum(flat * flat, axis=1, keepdims=True)
    cn = jnp.sum(codebook * codebook, axis=1)[None, :]

    idx = _k1(flat, codebook, zn, cn)
    idx2d = idx.reshape(n // 128, 128)
    zq = _make_k2(n, d, num_codes)(idx2d, codebook)
    zqst, loss, ppl = _k3(flat, zq, idx)
    return (zqst.reshape(b, nseq, d),
            loss.reshape(()),
            ppl.reshape(()))


# speedup vs baseline: 1.0909x; 1.0097x over previous
import functools

import jax
import jax.numpy as jnp
from jax import lax
from jax.experimental import pallas as pl
from jax.experimental.pallas import tpu as pltpu
from jax.experimental.pallas import tpu_sc as plsc

ALPHA = 0.25
BETA = 0.25

RB = 1024
CB = 2048


def _k1_body(x_ref, cb_ref, zn_ref, cn_ref, idx_ref, *, num_codes: int):
    xm2 = x_ref[...] * -2.0
    zn = zn_ref[...][:, 0]
    nchunks = num_codes // CB

    def step(j, carry):
        bd, bi = carry
        c = cb_ref[pl.ds(j * CB, CB), :]
        zc = lax.dot_general(
            xm2, c, (((1,), (1,)), ((), ())),
            preferred_element_type=jnp.float32,
            precision=lax.Precision.DEFAULT)

        cn = cn_ref[0, pl.ds(j * CB, CB)]
        s = (zn[:, None] + zc) + cn[None, :]
        m = jnp.min(s, axis=1)
        ii = lax.broadcasted_iota(jnp.int32, (RB, CB), 1)
        cand = jnp.where(s == m[:, None], ii, num_codes)
        a = jnp.min(cand, axis=1) + j * CB
        take = (m < bd) | ((m == bd) & (a < bi))
        nbd = jnp.where(take, m, bd)
        nbd = nbd.astype(jnp.bfloat16).astype(jnp.float32)
        return nbd, jnp.where(take, a, bi)

    init = (jnp.full((RB,), jnp.inf, jnp.float32),
            jnp.zeros((RB,), jnp.int32))
    _, bi = lax.fori_loop(0, nchunks, step, init)
    idx_ref[...] = bi


def _k1(flat, codebook, zn, cn):
    n, d = flat.shape
    num_codes = codebook.shape[0]
    grid = (n // RB,)
    return pl.pallas_call(
        functools.partial(_k1_body, num_codes=num_codes),
        grid=grid,
        in_specs=[
            pl.BlockSpec((RB, d), lambda i: (i, 0)),
            pl.BlockSpec((num_codes, d), lambda i: (0, 0)),
            pl.BlockSpec((RB, 1), lambda i: (i, 0)),
            pl.BlockSpec((1, num_codes), lambda i: (0, 0)),
        ],
        out_specs=pl.BlockSpec((RB,), lambda i: (i,)),
        out_shape=jax.ShapeDtypeStruct((n,), jnp.int32),
    )(flat, codebook, zn, cn)


K4_CB = 2048


K4_RB = 512


_NC, _NS, _L = 2, 16, 16
_NW = _NC * _NS
_ROWS_PER_TXN = 128


def _make_k2(n, d, num_codes):
    bpw = n // _NW
    ntxn = bpw // _ROWS_PER_TXN
    mesh = plsc.VectorSubcoreMesh(core_axis_name="c", subcore_axis_name="s")

    @functools.partial(
        pl.kernel,
        out_type=jax.ShapeDtypeStruct((n, d), jnp.float32),
        mesh=mesh,
        scratch_types=[
            [pltpu.VMEM((_ROWS_PER_TXN,), jnp.int32) for _ in range(ntxn)],
            [pltpu.VMEM((_ROWS_PER_TXN, d), jnp.float32) for _ in range(ntxn)],
            pltpu.SemaphoreType.DMA,
        ],
    )
    def k2(idx_hbm, cb_hbm, zq_hbm, idx_vs, rows_vs, sem):
        c = lax.axis_index("c")
        s = lax.axis_index("s")
        wid = s * _NC + c
        base = wid * bpw

        for j in range(ntxn):
            pltpu.sync_copy(idx_hbm.at[wid * ntxn + j], idx_vs[j])
        copies = [pltpu.async_copy(cb_hbm.at[idx_vs[j]], rows_vs[j], sem)
                  for j in range(ntxn)]
        for j in range(ntxn):
            copies[j].wait()
            pltpu.sync_copy(rows_vs[j],
                            zq_hbm.at[pl.ds(base + j * _ROWS_PER_TXN,
                                            _ROWS_PER_TXN)])

    return k2


K3_RB = 1024


def _k3_body(ze_ref, zq_ref, idx_ref, zqst_ref, loss_ref, ppl_ref,
             acc_ref, ent_ref, *, nsteps: int, n: int, d: int,
             num_codes: int):
    i = pl.program_id(0)
    ze = ze_ref[...]
    zq = zq_ref[...]
    zqst_ref[...] = ze + (zq - ze)
    blk = jnp.sum((ze - zq) ** 2)

    ccs = num_codes // nsteps
    codes = lax.broadcasted_iota(jnp.int32, (ccs, 512), 0) + i * ccs
    cnt = jnp.zeros((ccs,), jnp.float32)
    for k in range(n // 512):
        bi = idx_ref[pl.ds(k * 512, 512)]
        cnt = cnt + jnp.sum((codes == bi[None, :]).astype(jnp.float32), axis=1)
    p = cnt * (1.0 / jnp.float32(n))
    ent_blk = jnp.sum(p * jnp.log(p + 1e-10))

    @pl.when(i == 0)
    def _():
        acc_ref[0, 0] = blk
        ent_ref[0, 0] = ent_blk

    @pl.when(i > 0)
    def _():
        acc_ref[0, 0] = acc_ref[0, 0] + blk
        ent_ref[0, 0] = ent_ref[0, 0] + ent_blk

    @pl.when(i == nsteps - 1)
    def _():
        m = acc_ref[0, 0] / jnp.float32(n * d)
        loss_ref[...] = jnp.reshape(ALPHA * (m + BETA * m), (1, 1))
        ppl_ref[...] = jnp.reshape(jnp.exp(-(ent_ref[0, 0])), (1, 1))


def _k3(flat, zq, idx):
    n, d = flat.shape
    num_codes = 8192
    nsteps = n // K3_RB
    return pl.pallas_call(
        functools.partial(_k3_body, nsteps=nsteps, n=n, d=d,
                          num_codes=num_codes),
        grid=(nsteps,),
        in_specs=[
            pl.BlockSpec((K3_RB, d), lambda i: (i, 0)),
            pl.BlockSpec((K3_RB, d), lambda i: (i, 0)),
            pl.BlockSpec((n,), lambda i: (0,)),
        ],
        out_specs=[
            pl.BlockSpec((K3_RB, d), lambda i: (i, 0)),
            pl.BlockSpec((1, 1), lambda i: (0, 0)),
            pl.BlockSpec((1, 1), lambda i: (0, 0)),
        ],
        out_shape=[
            jax.ShapeDtypeStruct((n, d), jnp.float32),
            jax.ShapeDtypeStruct((1, 1), jnp.float32),
            jax.ShapeDtypeStruct((1, 1), jnp.float32),
        ],
        scratch_shapes=[pltpu.SMEM((1, 1), jnp.float32),
                        pltpu.SMEM((1, 1), jnp.float32)],
    )(flat, zq, idx)


def kernel(z_e, codebook):
    b, nseq, d = z_e.shape
    n = b * nseq
    num_codes = codebook.shape[0]
    flat = z_e.reshape(n, d)
    zn = jnp.sum(flat * flat, axis=1, keepdims=True)
    cn = jnp.sum(codebook * codebook, axis=1)[None, :]

    idx = _k1(flat, codebook, zn, cn)
    idx2d = idx.reshape(n // 128, 128)
    zq = _make_k2(n, d, num_codes)(idx2d, codebook)
    zqst, loss, ppl = _k3(flat, zq, idx)
    return (zqst.reshape(b, nseq, d),
            loss.reshape(()),
            ppl.reshape(()))
